# Initial kernel scaffold; baseline (speedup 1.0000x reference)
import jax
import jax.numpy as jnp
from jax.experimental import pallas as pl


def kernel(global_obs, adj, agent_id, W_enc, b_enc, W_l, b_l, W_r, b_r, att, b_gat, W_act, b_act, W_val, b_val):
    raise NotImplementedError("write your pallas kernel here")


# speedup vs baseline: 4.4727x; 4.4727x over previous
import functools

import jax
import jax.numpy as jnp
from jax.experimental import pallas as pl
from jax.experimental.pallas import tpu as pltpu


def _gappo_kernel(agent_ref, obs_ref, adj_ref, W_enc_ref, b_enc_ref,
                  W_l_ref, b_l_ref, W_r_ref, b_r_ref, att_ref, b_gat_ref,
                  W_act_ref, b_act_ref, W_val_ref, b_val_ref,
                  logits_ref, vs_ref, *, N, D, H):
    b = pl.program_id(0)
    agent = agent_ref[b, 0]

    obs = obs_ref[0]
    h = jnp.dot(obs, W_enc_ref[...], preferred_element_type=jnp.float32)
    h = jnp.maximum(h + b_enc_ref[...], 0.0)
    gl = jnp.dot(h, W_l_ref[...], preferred_element_type=jnp.float32) + b_l_ref[...]

    onehot = (jax.lax.broadcasted_iota(jnp.int32, (1, N), 1) == agent).astype(jnp.float32)
    h_a = jnp.dot(onehot, h, preferred_element_type=jnp.float32)
    gr = jnp.dot(h_a, W_r_ref[...], preferred_element_type=jnp.float32) + b_r_ref[...]

    adjv = adj_ref[0]
    adj_col = jnp.sum(jnp.where(adjv != 0, 1.0, 0.0) * onehot, axis=1, keepdims=True)
    jidx = jax.lax.broadcasted_iota(jnp.int32, (N, 1), 0)
    mask = (adj_col > 0.0) | (jidx == agent)

    acc = jnp.zeros((1, D), dtype=jnp.float32)
    for hh in range(H):
        glh = gl[:, hh * D:(hh + 1) * D]
        grh = gr[:, hh * D:(hh + 1) * D]
        e = glh + grh
        e = jnp.where(e >= 0.0, e, 0.2 * e)
        s = jnp.sum(e * att_ref[hh:hh + 1, :], axis=1, keepdims=True)
        s = jnp.where(mask, s, jnp.float32(-1e9))
        m = jnp.max(s, axis=0, keepdims=True)
        ex = jnp.exp(s - m)
        alpha = ex / jnp.sum(ex, axis=0, keepdims=True)
        acc = acc + jnp.sum(alpha * glh, axis=0, keepdims=True)

    out_a = acc * (1.0 / H) + b_gat_ref[...]
    feat = jnp.concatenate([h_a, out_a], axis=1)
    logits_ref[0] = jnp.dot(feat, W_act_ref[...], preferred_element_type=jnp.float32) + b_act_ref[...]
    vs_ref[0] = jnp.dot(feat, W_val_ref[...], preferred_element_type=jnp.float32) + b_val_ref[...]


@jax.jit
def kernel(global_obs, adj, agent_id, W_enc, b_enc, W_l, b_l, W_r, b_r, att,
           b_gat, W_act, b_act, W_val, b_val):
    B, N, F = global_obs.shape
    D = W_enc.shape[1]
    H = att.shape[0]
    A = W_act.shape[1]

    b_enc2 = b_enc.reshape(1, D)
    b_l2 = b_l.reshape(1, H * D)
    b_r2 = b_r.reshape(1, H * D)
    b_gat2 = b_gat.reshape(1, D)
    b_act2 = b_act.reshape(1, A)
    b_val2 = b_val.reshape(1, 1)

    full = lambda shape: pl.BlockSpec(shape, lambda b: (0,) * len(shape))
    logits, vs = pl.pallas_call(
        functools.partial(_gappo_kernel, N=N, D=D, H=H),
        grid=(B,),
        in_specs=[
            pl.BlockSpec(memory_space=pltpu.SMEM),
            pl.BlockSpec((1, N, F), lambda b: (b, 0, 0)),
            pl.BlockSpec((1, N, N), lambda b: (b, 0, 0)),
            full((F, D)), full((1, D)),
            full((D, H * D)), full((1, H * D)),
            full((D, H * D)), full((1, H * D)),
            full((H, D)), full((1, D)),
            full((2 * D, A)), full((1, A)),
            full((2 * D, 1)), full((1, 1)),
        ],
        out_specs=[
            pl.BlockSpec((1, 1, A), lambda b: (b, 0, 0)),
            pl.BlockSpec((1, 1, 1), lambda b: (b, 0, 0)),
        ],
        out_shape=[
            jax.ShapeDtypeStruct((B, 1, A), jnp.float32),
            jax.ShapeDtypeStruct((B, 1, 1), jnp.float32),
        ],
        compiler_params=pltpu.CompilerParams(
            dimension_semantics=("arbitrary",),
        ),
    )(agent_id, global_obs, adj, W_enc, b_enc2, W_l, b_l2, W_r, b_r2, att,
      b_gat2, W_act, b_act2, W_val, b_val2)
    return (logits.reshape(B, A), vs.reshape(B, 1))


# speedup vs baseline: 5.2926x; 1.1833x over previous
import functools

import jax
import jax.numpy as jnp
from jax.experimental import pallas as pl
from jax.experimental.pallas import tpu as pltpu


def _gappo_kernel(agent_ref, obs_ref, adj_ref, W_enc_ref, b_enc_ref,
                  W_l_ref, b_l_ref, W_r_ref, b_r_ref, att_ref, b_gat_ref,
                  W_act_ref, b_act_ref, W_val_ref, b_val_ref,
                  logits_ref, vs_ref, *, N, D, H, SB):
    g = pl.program_id(0)
    R = SB * N
    f32 = jnp.float32

    obs = obs_ref[...]
    h = jnp.dot(obs, W_enc_ref[...], preferred_element_type=f32)
    h = jnp.maximum(h + b_enc_ref[...], 0.0)
    gl = jnp.dot(h, W_l_ref[...], preferred_element_type=f32) + b_l_ref[...]

    lane = jax.lax.broadcasted_iota(jnp.int32, (1, N), 1)
    A = jnp.concatenate(
        [(lane == agent_ref[g * SB + i, 0]).astype(f32) for i in range(SB)],
        axis=0)

    ridx = jax.lax.broadcasted_iota(jnp.int32, (R, 1), 0)
    bloc = ridx // N
    jmod = ridx % N
    Sb = (bloc == jax.lax.broadcasted_iota(jnp.int32, (R, SB), 1)).astype(f32)
    SbT = (jax.lax.broadcasted_iota(jnp.int32, (SB, R), 1) // N ==
           jax.lax.broadcasted_iota(jnp.int32, (SB, R), 0)).astype(f32)
    K = (jax.lax.broadcasted_iota(jnp.int32, (R, N), 1) == jmod).astype(f32)

    OnN = jnp.dot(Sb, A, preferred_element_type=f32)
    selfsel = jnp.sum(OnN * K, axis=1, keepdims=True)

    adjf = jnp.where(adj_ref[...] != 0, 1.0, 0.0)
    adj_col = jnp.sum(adjf * OnN, axis=1, keepdims=True)
    mask = (adj_col > 0.0) | (selfsel > 0.0)

    h_a = jnp.dot(SbT, h * selfsel, preferred_element_type=f32)
    gr = jnp.dot(h_a, W_r_ref[...], preferred_element_type=f32) + b_r_ref[...]
    Gexp = jnp.dot(Sb, gr, preferred_element_type=f32)

    E = gl + Gexp
    E = jnp.where(E >= 0.0, E, 0.2 * E)
    P = E * att_ref[...]

    alpha_cols = []
    for hh in range(H):
        s = jnp.sum(P[:, hh * D:(hh + 1) * D], axis=1, keepdims=True)
        s = jnp.where(mask, s, f32(-1e9))
        s2 = jnp.dot(SbT, s * K, preferred_element_type=f32)
        m = jnp.max(s2, axis=1, keepdims=True)
        ex = jnp.exp(s2 - m)
        alpha2 = ex / jnp.sum(ex, axis=1, keepdims=True)
        aflat = jnp.sum(jnp.dot(Sb, alpha2, preferred_element_type=f32) * K,
                        axis=1, keepdims=True)
        alpha_cols.append(jnp.broadcast_to(aflat, (R, D)))
    alpha_cat = jnp.concatenate(alpha_cols, axis=1)

    out_full = jnp.dot(SbT, alpha_cat * gl, preferred_element_type=f32)
    acc = out_full[:, 0:D]
    for hh in range(1, H):
        acc = acc + out_full[:, hh * D:(hh + 1) * D]
    out_mean = acc * (1.0 / H) + b_gat_ref[...]

    feat = jnp.concatenate([h_a, out_mean], axis=1)
    logits_ref[:, 0, :] = jnp.dot(feat, W_act_ref[...], preferred_element_type=f32) + b_act_ref[...]
    vs_ref[:, 0, :] = jnp.dot(feat, W_val_ref[...], preferred_element_type=f32) + b_val_ref[...]


@jax.jit
def kernel(global_obs, adj, agent_id, W_enc, b_enc, W_l, b_l, W_r, b_r, att,
           b_gat, W_act, b_act, W_val, b_val):
    B, N, F = global_obs.shape
    D = W_enc.shape[1]
    H = att.shape[0]
    A = W_act.shape[1]
    SB = 4
    R = SB * N

    obs2d = global_obs.reshape(B * N, F)
    adj2d = adj.reshape(B * N, N)
    att2 = att.reshape(1, H * D)
    b_enc2 = b_enc.reshape(1, D)
    b_l2 = b_l.reshape(1, H * D)
    b_r2 = b_r.reshape(1, H * D)
    b_gat2 = b_gat.reshape(1, D)
    b_act2 = b_act.reshape(1, A)
    b_val2 = b_val.reshape(1, 1)

    full = lambda shape: pl.BlockSpec(shape, lambda g: (0,) * len(shape))
    logits, vs = pl.pallas_call(
        functools.partial(_gappo_kernel, N=N, D=D, H=H, SB=SB),
        grid=(B // SB,),
        in_specs=[
            pl.BlockSpec(memory_space=pltpu.SMEM),
            pl.BlockSpec((R, F), lambda g: (g, 0)),
            pl.BlockSpec((R, N), lambda g: (g, 0)),
            full((F, D)), full((1, D)),
            full((D, H * D)), full((1, H * D)),
            full((D, H * D)), full((1, H * D)),
            full((1, H * D)), full((1, D)),
            full((2 * D, A)), full((1, A)),
            full((2 * D, 1)), full((1, 1)),
        ],
        out_specs=[
            pl.BlockSpec((SB, 1, A), lambda g: (g, 0, 0)),
            pl.BlockSpec((SB, 1, 1), lambda g: (g, 0, 0)),
        ],
        out_shape=[
            jax.ShapeDtypeStruct((B, 1, A), jnp.float32),
            jax.ShapeDtypeStruct((B, 1, 1), jnp.float32),
        ],
        compiler_params=pltpu.CompilerParams(
            dimension_semantics=("arbitrary",),
        ),
    )(agent_id, obs2d, adj2d, W_enc, b_enc2, W_l, b_l2, W_r, b_r2, att2,
      b_gat2, W_act, b_act2, W_val, b_val2)
    return (logits.reshape(B, A), vs.reshape(B, 1))


# speedup vs baseline: 5.3275x; 1.0066x over previous
import functools

import jax
import jax.numpy as jnp
from jax.experimental import pallas as pl
from jax.experimental.pallas import tpu as pltpu


def _gappo_kernel(agent_ref, obs_ref, adj_ref, W_enc_ref, b_enc_ref,
                  W_l_ref, b_l_ref, W_r_ref, b_r_ref, attblk_ref, b_gat_ref,
                  W_act_ref, b_act_ref, W_val_ref, b_val_ref,
                  logits_ref, vs_ref, *, N, D, H, SB):
    g = pl.program_id(0)
    R = SB * N
    f32 = jnp.float32
    i32 = jnp.int32
    iota = jax.lax.broadcasted_iota

    obs = obs_ref[...]
    h = jnp.dot(obs, W_enc_ref[...], preferred_element_type=f32)
    h = jnp.maximum(h + b_enc_ref[...], 0.0)
    gl = jnp.dot(h, W_l_ref[...], preferred_element_type=f32) + b_l_ref[...]

    agents = [agent_ref[g * SB + i, 0] for i in range(SB)]
    agar = jnp.concatenate(
        [jnp.full((N, 1), a, dtype=i32) for a in agents], axis=0)
    AT = jnp.concatenate(
        [(iota(i32, (N, 1), 0) == a).astype(f32) for a in agents], axis=1)

    ridx = iota(i32, (R, 1), 0)
    jmod = ridx % N
    Sb = (ridx // N == iota(i32, (R, SB), 1)).astype(f32)
    SbT = (iota(i32, (SB, R), 1) // N ==
           iota(i32, (SB, R), 0)).astype(f32)

    selfsel_b = jmod == agar
    selfsel = selfsel_b.astype(f32)

    adjf = (adj_ref[...] != 0).astype(f32)
    Cm = jnp.dot(adjf, AT, preferred_element_type=f32)
    adj_col = jnp.sum(Cm * Sb, axis=1, keepdims=True)
    mask = (adj_col > 0.0) | selfsel_b

    h_a = jnp.dot(SbT, h * selfsel, preferred_element_type=f32)
    gr = jnp.dot(h_a, W_r_ref[...], preferred_element_type=f32) + b_r_ref[...]
    Gexp = jnp.dot(Sb, gr, preferred_element_type=f32)

    E = gl + Gexp
    E = jnp.where(E >= 0.0, E, 0.2 * E)
    s_both = jnp.dot(E, attblk_ref[...], preferred_element_type=f32)
    s_both = jnp.where(mask, s_both, f32(-1e9))

    K = (iota(i32, (R, N), 1) == jmod).astype(f32)
    KT = (iota(i32, (N, R), 0) ==
          iota(i32, (N, R), 1) % N).astype(f32)

    alpha_cols = []
    for hh in range(H):
        sh = s_both[:, hh:hh + 1]
        s2T = jnp.dot(KT, sh * Sb, preferred_element_type=f32)
        m = jnp.max(s2T, axis=0, keepdims=True)
        ex = jnp.exp(s2T - m)
        alphaT = ex / jnp.sum(ex, axis=0, keepdims=True)
        af_all = jnp.dot(K, alphaT, preferred_element_type=f32)
        aflat = jnp.sum(af_all * Sb, axis=1, keepdims=True)
        alpha_cols.append(jnp.broadcast_to(aflat, (R, D)))
    alpha_cat = jnp.concatenate(alpha_cols, axis=1)

    out_full = jnp.dot(SbT, alpha_cat * gl, preferred_element_type=f32)
    acc = out_full[:, 0:D]
    for hh in range(1, H):
        acc = acc + out_full[:, hh * D:(hh + 1) * D]
    out_mean = acc * (1.0 / H) + b_gat_ref[...]

    feat = jnp.concatenate([h_a, out_mean], axis=1)
    logits_ref[:, 0, :] = jnp.dot(feat, W_act_ref[...], preferred_element_type=f32) + b_act_ref[...]
    vs_ref[:, 0, :] = jnp.dot(feat, W_val_ref[...], preferred_element_type=f32) + b_val_ref[...]


@jax.jit
def kernel(global_obs, adj, agent_id, W_enc, b_enc, W_l, b_l, W_r, b_r, att,
           b_gat, W_act, b_act, W_val, b_val):
    B, N, F = global_obs.shape
    D = W_enc.shape[1]
    H = att.shape[0]
    A = W_act.shape[1]
    SB = 4
    R = SB * N

    obs2d = global_obs.reshape(B * N, F)
    adj2d = adj.reshape(B * N, N)
    attblk = (att[:, :, None] * jnp.eye(H, dtype=att.dtype)[:, None, :]).reshape(H * D, H)
    b_enc2 = b_enc.reshape(1, D)
    b_l2 = b_l.reshape(1, H * D)
    b_r2 = b_r.reshape(1, H * D)
    b_gat2 = b_gat.reshape(1, D)
    b_act2 = b_act.reshape(1, A)
    b_val2 = b_val.reshape(1, 1)

    full = lambda shape: pl.BlockSpec(shape, lambda g: (0,) * len(shape))
    logits, vs = pl.pallas_call(
        functools.partial(_gappo_kernel, N=N, D=D, H=H, SB=SB),
        grid=(B // SB,),
        in_specs=[
            pl.BlockSpec(memory_space=pltpu.SMEM),
            pl.BlockSpec((R, F), lambda g: (g, 0)),
            pl.BlockSpec((R, N), lambda g: (g, 0)),
            full((F, D)), full((1, D)),
            full((D, H * D)), full((1, H * D)),
            full((D, H * D)), full((1, H * D)),
            full((H * D, H)), full((1, D)),
            full((2 * D, A)), full((1, A)),
            full((2 * D, 1)), full((1, 1)),
        ],
        out_specs=[
            pl.BlockSpec((SB, 1, A), lambda g: (g, 0, 0)),
            pl.BlockSpec((SB, 1, 1), lambda g: (g, 0, 0)),
        ],
        out_shape=[
            jax.ShapeDtypeStruct((B, 1, A), jnp.float32),
            jax.ShapeDtypeStruct((B, 1, 1), jnp.float32),
        ],
        compiler_params=pltpu.CompilerParams(
            dimension_semantics=("arbitrary",),
        ),
    )(agent_id, obs2d, adj2d, W_enc, b_enc2, W_l, b_l2, W_r, b_r2, attblk,
      b_gat2, W_act, b_act2, W_val, b_val2)
    return (logits.reshape(B, A), vs.reshape(B, 1))


# speedup vs baseline: 5.8127x; 1.0911x over previous
import functools

import jax
import jax.numpy as jnp
from jax.experimental import pallas as pl
from jax.experimental.pallas import tpu as pltpu


def _gappo_kernel(agent_ref, obs_ref, adj_ref, W_enc_ref, b_enc_ref,
                  W_l_ref, b_l_ref, W_r_ref, b_r_ref, attblk_ref, b_gat_ref,
                  W_act_ref, b_act_ref, W_val_ref, b_val_ref,
                  logits_ref, vs_ref, *, N, D, H, SB):
    g = pl.program_id(0)
    R = SB * N
    f32 = jnp.float32
    i32 = jnp.int32
    iota = jax.lax.broadcasted_iota

    obs = obs_ref[...]
    h = jnp.dot(obs, W_enc_ref[...], preferred_element_type=f32)
    h = jnp.maximum(h + b_enc_ref[...], 0.0)
    gl = jnp.dot(h, W_l_ref[...], preferred_element_type=f32) + b_l_ref[...]

    agents = [agent_ref[g * SB + i, 0] for i in range(SB)]
    agar = jnp.concatenate(
        [jnp.full((N, 1), a, dtype=i32) for a in agents], axis=0)
    AT = jnp.concatenate(
        [(iota(i32, (N, 1), 0) == a).astype(f32) for a in agents], axis=1)

    ridx = iota(i32, (R, 1), 0)
    jmod = ridx % N
    Sb = (ridx // N == iota(i32, (R, SB), 1)).astype(f32)
    SbT = (iota(i32, (SB, R), 1) // N ==
           iota(i32, (SB, R), 0)).astype(f32)

    selfsel_b = jmod == agar
    selfsel = selfsel_b.astype(f32)

    adjf = (adj_ref[...] != 0).astype(f32)
    Cm = jnp.dot(adjf, AT, preferred_element_type=f32)
    adj_col = jnp.sum(Cm * Sb, axis=1, keepdims=True)
    mask = (adj_col > 0.0) | selfsel_b

    h_a = jnp.dot(SbT, h * selfsel, preferred_element_type=f32)
    gr = jnp.dot(h_a, W_r_ref[...], preferred_element_type=f32) + b_r_ref[...]
    Gexp = jnp.dot(Sb, gr, preferred_element_type=f32)

    E = gl + Gexp
    E = jnp.where(E >= 0.0, E, 0.2 * E)
    s_both = jnp.dot(E, attblk_ref[...], preferred_element_type=f32)
    s_both = jnp.where(mask, s_both, f32(-1e9))

    K = (iota(i32, (R, N), 1) == jmod).astype(f32)
    KT = (iota(i32, (N, R), 0) ==
          iota(i32, (N, R), 1) % N).astype(f32)

    alpha_cols = []
    for hh in range(H):
        sh = s_both[:, hh:hh + 1]
        s2T = jnp.dot(KT, sh * Sb, preferred_element_type=f32)
        m = jnp.max(s2T, axis=0, keepdims=True)
        ex = jnp.exp(s2T - m)
        alphaT = ex / jnp.sum(ex, axis=0, keepdims=True)
        af_all = jnp.dot(K, alphaT, preferred_element_type=f32)
        aflat = jnp.sum(af_all * Sb, axis=1, keepdims=True)
        alpha_cols.append(jnp.broadcast_to(aflat, (R, D)))
    alpha_cat = jnp.concatenate(alpha_cols, axis=1)

    out_full = jnp.dot(SbT, alpha_cat * gl, preferred_element_type=f32)
    acc = out_full[:, 0:D]
    for hh in range(1, H):
        acc = acc + out_full[:, hh * D:(hh + 1) * D]
    out_mean = acc * (1.0 / H) + b_gat_ref[...]

    feat = jnp.concatenate([h_a, out_mean], axis=1)
    logits_ref[:, 0, :] = jnp.dot(feat, W_act_ref[...], preferred_element_type=f32) + b_act_ref[...]
    vs_ref[:, 0, :] = jnp.dot(feat, W_val_ref[...], preferred_element_type=f32) + b_val_ref[...]


@jax.jit
def kernel(global_obs, adj, agent_id, W_enc, b_enc, W_l, b_l, W_r, b_r, att,
           b_gat, W_act, b_act, W_val, b_val):
    B, N, F = global_obs.shape
    D = W_enc.shape[1]
    H = att.shape[0]
    A = W_act.shape[1]
    SB = 8
    R = SB * N

    obs2d = global_obs.reshape(B * N, F)
    adj2d = adj.reshape(B * N, N)
    attblk = (att[:, :, None] * jnp.eye(H, dtype=att.dtype)[:, None, :]).reshape(H * D, H)
    b_enc2 = b_enc.reshape(1, D)
    b_l2 = b_l.reshape(1, H * D)
    b_r2 = b_r.reshape(1, H * D)
    b_gat2 = b_gat.reshape(1, D)
    b_act2 = b_act.reshape(1, A)
    b_val2 = b_val.reshape(1, 1)

    full = lambda shape: pl.BlockSpec(shape, lambda g: (0,) * len(shape))
    logits, vs = pl.pallas_call(
        functools.partial(_gappo_kernel, N=N, D=D, H=H, SB=SB),
        grid=(B // SB,),
        in_specs=[
            pl.BlockSpec(memory_space=pltpu.SMEM),
            pl.BlockSpec((R, F), lambda g: (g, 0)),
            pl.BlockSpec((R, N), lambda g: (g, 0)),
            full((F, D)), full((1, D)),
            full((D, H * D)), full((1, H * D)),
            full((D, H * D)), full((1, H * D)),
            full((H * D, H)), full((1, D)),
            full((2 * D, A)), full((1, A)),
            full((2 * D, 1)), full((1, 1)),
        ],
        out_specs=[
            pl.BlockSpec((SB, 1, A), lambda g: (g, 0, 0)),
            pl.BlockSpec((SB, 1, 1), lambda g: (g, 0, 0)),
        ],
        out_shape=[
            jax.ShapeDtypeStruct((B, 1, A), jnp.float32),
            jax.ShapeDtypeStruct((B, 1, 1), jnp.float32),
        ],
        compiler_params=pltpu.CompilerParams(
            dimension_semantics=("arbitrary",),
        ),
    )(agent_id, obs2d, adj2d, W_enc, b_enc2, W_l, b_l2, W_r, b_r2, attblk,
      b_gat2, W_act, b_act2, W_val, b_val2)
    return (logits.reshape(B, A), vs.reshape(B, 1))
